# baseline (speedup 1.0000x reference)
import functools

import numpy as np

import jax
import jax.numpy as jnp
from jax import lax
from jax.experimental import pallas as pl
from jax.experimental.pallas import tpu as pltpu
from jax.experimental.pallas import tpu_sc as plsc

_PREC = lax.Precision.HIGHEST


def _dot(a, b):
    return jnp.dot(a, b, preferred_element_type=jnp.float32, precision=_PREC)


def _ner_body(x_ref, w1_ref, b1_ref, w2_ref, b2_ref, out_ref):
    h = jnp.maximum(_dot(x_ref[...], w1_ref[...]) + b1_ref[...], 0.0)
    out_ref[...] = _dot(h, w2_ref[...]) + b2_ref[...]


def _ner_mlp(x2, w1, b1, w2, b2, block_m=1024):
    n_rows, d = x2.shape
    nd = w2.shape[1]
    grid = (n_rows // block_m,)
    return pl.pallas_call(
        _ner_body,
        grid=grid,
        in_specs=[
            pl.BlockSpec((block_m, d), lambda i: (i, 0)),
            pl.BlockSpec((d, d), lambda i: (0, 0)),
            pl.BlockSpec((1, d), lambda i: (0, 0)),
            pl.BlockSpec((d, nd), lambda i: (0, 0)),
            pl.BlockSpec((1, nd), lambda i: (0, 0)),
        ],
        out_specs=pl.BlockSpec((block_m, nd), lambda i: (i, 0)),
        out_shape=jax.ShapeDtypeStruct((n_rows, nd), jnp.float32),
    )(x2, w1, b1, w2, b2)


def _sc_gather(x2, idx_flat, ent_lim, seq_len):
    n_idx = idx_flat.shape[0]
    d = x2.shape[1]
    rows_per_w = 16
    n_workers = n_idx // rows_per_w

    mesh = plsc.VectorSubcoreMesh(core_axis_name="c", subcore_axis_name="s")

    @functools.partial(
        pl.kernel,
        mesh=mesh,
        out_type=jax.ShapeDtypeStruct((n_idx, d), jnp.float32),
        scratch_types=[
            pltpu.VMEM((rows_per_w,), jnp.int32),
            pltpu.VMEM((rows_per_w, d), jnp.float32),
            pltpu.SemaphoreType.DMA,
        ],
    )
    def _gather(hid_hbm, idx_hbm, out_hbm, idx_v, rows_v, sem):
        wid = lax.axis_index("s") * 2 + lax.axis_index("c")

        @pl.when(wid < n_workers)
        def _():
            base = wid * rows_per_w
            pltpu.sync_copy(idx_hbm.at[pl.ds(base, rows_per_w)], idx_v)
            g = lax.iota(jnp.int32, 16) + base
            scale = (1 << 16) // ent_lim + 1
            b = lax.shift_right_logical(g * scale, 16)
            idx_v[...] = idx_v[...] + b * seq_len + 1
            pltpu.async_copy(hid_hbm.at[idx_v], rows_v, sem).wait()
            pltpu.sync_copy(rows_v, out_hbm.at[pl.ds(base, rows_per_w)])

    return _gather(x2, idx_flat)


def _re_body(re_dim, ent_lim,
             ents_ref, posf_ref, wn1_ref, bn1_ref, wn2_ref, bn2_ref,
             wh1a_ref, wh1b_ref, bh1_ref, wh2_ref, bh2_ref,
             wt1a_ref, wt1b_ref, bt1_ref, wt2_ref, bt2_ref,
             wbil_ref, bbil_ref, wlh_ref, wlt_ref,
             rall_ref, call_ref, mbd_ref, ffold_ref, mt_ref, f2_ref,
             re_out_ref, pos_out_ref):
    x = ents_ref[...]
    hn = jnp.maximum(_dot(x, wn1_ref[...]) + bn1_ref[...], 0.0)
    nerr = _dot(hn, wn2_ref[...]) + bn2_ref[...]
    p = jax.nn.softmax(nerr, axis=-1)

    h = jnp.maximum(_dot(x, wh1a_ref[...]) + _dot(p, wh1b_ref[...])
                    + bh1_ref[...], 0.0)
    head = _dot(h, wh2_ref[...]) + bh2_ref[...]
    t = jnp.maximum(_dot(x, wt1a_ref[...]) + _dot(p, wt1b_ref[...])
                    + bt1_ref[...], 0.0)
    tail = _dot(t, wt2_ref[...]) + bt2_ref[...]

    mbd = mbd_ref[...]
    ffold = ffold_ref[...]
    cols = []
    for o in range(re_dim):
        a_o = _dot(head, wbil_ref[o])
        g_o = lax.dot_general(a_o, tail, (((1,), (1,)), ((), ())),
                              preferred_element_type=jnp.float32,
                              precision=_PREC)
        cols.append(_dot(g_o * mbd, ffold))
    bilp = jnp.concatenate(cols, axis=1) + bbil_ref[...]

    head_l = _dot(head, wlh_ref[...])
    tail_l = _dot(tail, wlt_ref[...])

    rall = rall_ref[...]
    callm = call_ref[...]
    s1 = _dot(rall, bilp) * mt_ref[...]
    bilsel = _dot(s1, f2_ref[...])
    linsel = _dot(rall, head_l) + _dot(callm, tail_l)
    re_out_ref[...] = bilsel + linsel

    posf = posf_ref[...]
    ph = _dot(rall, posf)
    pt = _dot(callm, posf)
    pos_out_ref[...] = jnp.concatenate([ph, pt], axis=1).astype(jnp.int32)


def _re_head(ents, posf, wn1, bn1, wn2, bn2, wh1a, wh1b, bh1, wh2, bh2,
             wt1a, wt1b, bt1, wt2, bt2, wbil, bbil_exp, wlh, wlt,
             rall, callm, mbd, ffold, mt, f2):
    n_pair_rows = rall.shape[0]
    re_dim = wbil.shape[0]
    ent_lim = ffold.shape[1]
    body = functools.partial(_re_body, re_dim, ent_lim)
    return pl.pallas_call(
        body,
        out_shape=(
            jax.ShapeDtypeStruct((n_pair_rows, re_dim), jnp.float32),
            jax.ShapeDtypeStruct((n_pair_rows, 2), jnp.int32),
        ),
    )(ents, posf, wn1, bn1, wn2, bn2, wh1a, wh1b, bh1, wh2, bh2,
      wt1a, wt1b, bt1, wt2, bt2, wbil, bbil_exp, wlh, wlt,
      rall, callm, mbd, ffold, mt, f2)


def kernel(hidden, entity_idx, W_ner1, b_ner1, W_ner2, b_ner2,
           W_h1, b_h1, W_h2, b_h2, W_t1, b_t1, W_t2, b_t2,
           W_bil, b_bil, W_lin):
    B, S, D = hidden.shape
    L = entity_idx.shape[1]
    nd = W_ner2.shape[1]
    h_dim = W_h2.shape[0]
    re_dim = W_bil.shape[0]
    P = L * (L - 1)

    x2 = hidden.reshape(B * S, D)

    ner = _ner_mlp(x2, W_ner1, b_ner1.reshape(1, D),
                   W_ner2, b_ner2.reshape(1, nd))

    ents = _sc_gather(x2, entity_idx.reshape(-1), L, S)

    prs = [(i, j) for i in range(L) for j in range(L) if j != i]
    ih = np.array([q[0] for q in prs], dtype=np.int64)
    it = np.array([q[1] for q in prs], dtype=np.int64)

    rall = np.zeros((B * P, B * L), dtype=np.float32)
    callm = np.zeros((B * P, B * L), dtype=np.float32)
    bb = np.repeat(np.arange(B), P) * L
    rall[np.arange(B * P), bb + np.tile(ih, B)] = 1.0
    callm[np.arange(B * P), bb + np.tile(it, B)] = 1.0

    xg = np.arange(B * L) // L
    mbd = (xg[:, None] == xg[None, :]).astype(np.float32)
    ffold = (np.arange(B * L)[:, None] % L ==
             np.arange(L)[None, :]).astype(np.float32)
    mt90 = np.zeros((P, re_dim * L), dtype=np.float32)
    mt90[np.arange(P)[:, None],
         np.arange(re_dim)[None, :] * L + it[:, None]] = 1.0
    mt = np.tile(mt90, (B, 1))
    f2 = (np.arange(re_dim * L)[:, None] // L ==
          np.arange(re_dim)[None, :]).astype(np.float32)

    posf = (entity_idx.reshape(B * L, 1) + 1).astype(jnp.float32)
    bbil_exp = jnp.repeat(b_bil, L).reshape(1, re_dim * L)

    re_out, pos_out = _re_head(
        ents, posf,
        W_ner1, b_ner1.reshape(1, D), W_ner2, b_ner2.reshape(1, nd),
        W_h1[:D], W_h1[D:], b_h1.reshape(1, h_dim), W_h2,
        b_h2.reshape(1, h_dim),
        W_t1[:D], W_t1[D:], b_t1.reshape(1, h_dim), W_t2,
        b_t2.reshape(1, h_dim),
        W_bil, bbil_exp, W_lin[:h_dim], W_lin[h_dim:],
        jnp.asarray(rall), jnp.asarray(callm), jnp.asarray(mbd),
        jnp.asarray(ffold), jnp.asarray(mt), jnp.asarray(f2))

    return (ner.reshape(B, S, nd),
            pos_out.reshape(B, P, 2),
            re_out.reshape(B, P, re_dim))


# speedup vs baseline: 3.5060x; 3.5060x over previous
import functools

import numpy as np

import jax
import jax.numpy as jnp
from jax import lax
from jax.experimental import pallas as pl
from jax.experimental.pallas import tpu as pltpu
from jax.experimental.pallas import tpu_sc as plsc

_PREC = lax.Precision.DEFAULT


def _dot(a, b):
    return jnp.dot(a, b, preferred_element_type=jnp.float32, precision=_PREC)


def _ner_body(x_ref, w1_ref, b1_ref, w2_ref, b2_ref, out_ref):
    h = jnp.maximum(_dot(x_ref[...], w1_ref[...]) + b1_ref[...], 0.0)
    out_ref[...] = _dot(h, w2_ref[...]) + b2_ref[...]


def _ner_mlp(x2, w1, b1, w2, b2, block_m=1024):
    n_rows, d = x2.shape
    nd = w2.shape[1]
    grid = (n_rows // block_m,)
    return pl.pallas_call(
        _ner_body,
        grid=grid,
        in_specs=[
            pl.BlockSpec((block_m, d), lambda i: (i, 0)),
            pl.BlockSpec((d, d), lambda i: (0, 0)),
            pl.BlockSpec((1, d), lambda i: (0, 0)),
            pl.BlockSpec((d, nd), lambda i: (0, 0)),
            pl.BlockSpec((1, nd), lambda i: (0, 0)),
        ],
        out_specs=pl.BlockSpec((block_m, nd), lambda i: (i, 0)),
        out_shape=jax.ShapeDtypeStruct((n_rows, nd), jnp.float32),
    )(x2, w1, b1, w2, b2)


def _sc_gather(x2, idx_flat, ent_lim, seq_len):
    n_idx = idx_flat.shape[0]
    d = x2.shape[1]
    rows_per_w = 16
    n_workers = n_idx // rows_per_w

    mesh = plsc.VectorSubcoreMesh(core_axis_name="c", subcore_axis_name="s")

    @functools.partial(
        pl.kernel,
        mesh=mesh,
        out_type=jax.ShapeDtypeStruct((n_idx, d), jnp.float32),
        scratch_types=[
            pltpu.VMEM((rows_per_w,), jnp.int32),
            pltpu.VMEM((rows_per_w, d), jnp.float32),
            pltpu.SemaphoreType.DMA,
        ],
    )
    def _gather(hid_hbm, idx_hbm, out_hbm, idx_v, rows_v, sem):
        wid = lax.axis_index("s") * 2 + lax.axis_index("c")

        @pl.when(wid < n_workers)
        def _():
            base = wid * rows_per_w
            pltpu.sync_copy(idx_hbm.at[pl.ds(base, rows_per_w)], idx_v)
            g = lax.iota(jnp.int32, 16) + base
            scale = (1 << 16) // ent_lim + 1
            b = lax.shift_right_logical(g * scale, 16)
            idx_v[...] = idx_v[...] + b * seq_len + 1
            pltpu.async_copy(hid_hbm.at[idx_v], rows_v, sem).wait()
            pltpu.sync_copy(rows_v, out_hbm.at[pl.ds(base, rows_per_w)])

    return _gather(x2, idx_flat)


def _re_body(re_dim, ent_lim,
             ents_ref, posf_ref, wn1_ref, bn1_ref, wn2_ref, bn2_ref,
             wh1a_ref, wh1b_ref, bh1_ref, wh2_ref, bh2_ref,
             wt1a_ref, wt1b_ref, bt1_ref, wt2_ref, bt2_ref,
             wbil_ref, bbil_ref, wlh_ref, wlt_ref,
             rall_ref, call_ref, mbd_ref, ffold_ref, mt_ref, f2_ref,
             re_out_ref, pos_out_ref):
    x = ents_ref[...]
    hn = jnp.maximum(_dot(x, wn1_ref[...]) + bn1_ref[...], 0.0)
    nerr = _dot(hn, wn2_ref[...]) + bn2_ref[...]
    p = jax.nn.softmax(nerr, axis=-1)

    h = jnp.maximum(_dot(x, wh1a_ref[...]) + _dot(p, wh1b_ref[...])
                    + bh1_ref[...], 0.0)
    head = _dot(h, wh2_ref[...]) + bh2_ref[...]
    t = jnp.maximum(_dot(x, wt1a_ref[...]) + _dot(p, wt1b_ref[...])
                    + bt1_ref[...], 0.0)
    tail = _dot(t, wt2_ref[...]) + bt2_ref[...]

    mbd = mbd_ref[...]
    ffold = ffold_ref[...]
    cols = []
    for o in range(re_dim):
        a_o = _dot(head, wbil_ref[o])
        g_o = lax.dot_general(a_o, tail, (((1,), (1,)), ((), ())),
                              preferred_element_type=jnp.float32,
                              precision=_PREC)
        cols.append(_dot(g_o * mbd, ffold))
    bilp = jnp.concatenate(cols, axis=1) + bbil_ref[...]

    head_l = _dot(head, wlh_ref[...])
    tail_l = _dot(tail, wlt_ref[...])

    rall = rall_ref[...]
    callm = call_ref[...]
    s1 = _dot(rall, bilp) * mt_ref[...]
    bilsel = _dot(s1, f2_ref[...])
    linsel = _dot(rall, head_l) + _dot(callm, tail_l)
    re_out_ref[...] = bilsel + linsel

    posf = posf_ref[...]
    ph = jnp.dot(rall, posf, preferred_element_type=jnp.float32,
                 precision=lax.Precision.HIGHEST)
    pt = jnp.dot(callm, posf, preferred_element_type=jnp.float32,
                 precision=lax.Precision.HIGHEST)
    pos_out_ref[...] = jnp.concatenate([ph, pt], axis=1).astype(jnp.int32)


def _re_head(ents, posf, wn1, bn1, wn2, bn2, wh1a, wh1b, bh1, wh2, bh2,
             wt1a, wt1b, bt1, wt2, bt2, wbil, bbil_exp, wlh, wlt,
             rall, callm, mbd, ffold, mt, f2):
    n_pair_rows = rall.shape[0]
    re_dim = wbil.shape[0]
    ent_lim = ffold.shape[1]
    body = functools.partial(_re_body, re_dim, ent_lim)
    return pl.pallas_call(
        body,
        out_shape=(
            jax.ShapeDtypeStruct((n_pair_rows, re_dim), jnp.float32),
            jax.ShapeDtypeStruct((n_pair_rows, 2), jnp.int32),
        ),
    )(ents, posf, wn1, bn1, wn2, bn2, wh1a, wh1b, bh1, wh2, bh2,
      wt1a, wt1b, bt1, wt2, bt2, wbil, bbil_exp, wlh, wlt,
      rall, callm, mbd, ffold, mt, f2)


def kernel(hidden, entity_idx, W_ner1, b_ner1, W_ner2, b_ner2,
           W_h1, b_h1, W_h2, b_h2, W_t1, b_t1, W_t2, b_t2,
           W_bil, b_bil, W_lin):
    B, S, D = hidden.shape
    L = entity_idx.shape[1]
    nd = W_ner2.shape[1]
    h_dim = W_h2.shape[0]
    re_dim = W_bil.shape[0]
    P = L * (L - 1)

    x2 = hidden.reshape(B * S, D)

    ner = _ner_mlp(x2, W_ner1, b_ner1.reshape(1, D),
                   W_ner2, b_ner2.reshape(1, nd))

    ents = _sc_gather(x2, entity_idx.reshape(-1), L, S)

    prs = [(i, j) for i in range(L) for j in range(L) if j != i]
    ih = np.array([q[0] for q in prs], dtype=np.int64)
    it = np.array([q[1] for q in prs], dtype=np.int64)

    rall = np.zeros((B * P, B * L), dtype=np.float32)
    callm = np.zeros((B * P, B * L), dtype=np.float32)
    bb = np.repeat(np.arange(B), P) * L
    rall[np.arange(B * P), bb + np.tile(ih, B)] = 1.0
    callm[np.arange(B * P), bb + np.tile(it, B)] = 1.0

    xg = np.arange(B * L) // L
    mbd = (xg[:, None] == xg[None, :]).astype(np.float32)
    ffold = (np.arange(B * L)[:, None] % L ==
             np.arange(L)[None, :]).astype(np.float32)
    mt90 = np.zeros((P, re_dim * L), dtype=np.float32)
    mt90[np.arange(P)[:, None],
         np.arange(re_dim)[None, :] * L + it[:, None]] = 1.0
    mt = np.tile(mt90, (B, 1))
    f2 = (np.arange(re_dim * L)[:, None] // L ==
          np.arange(re_dim)[None, :]).astype(np.float32)

    posf = (entity_idx.reshape(B * L, 1) + 1).astype(jnp.float32)
    bbil_exp = jnp.repeat(b_bil, L).reshape(1, re_dim * L)

    re_out, pos_out = _re_head(
        ents, posf,
        W_ner1, b_ner1.reshape(1, D), W_ner2, b_ner2.reshape(1, nd),
        W_h1[:D], W_h1[D:], b_h1.reshape(1, h_dim), W_h2,
        b_h2.reshape(1, h_dim),
        W_t1[:D], W_t1[D:], b_t1.reshape(1, h_dim), W_t2,
        b_t2.reshape(1, h_dim),
        W_bil, bbil_exp, W_lin[:h_dim], W_lin[h_dim:],
        jnp.asarray(rall), jnp.asarray(callm), jnp.asarray(mbd),
        jnp.asarray(ffold), jnp.asarray(mt), jnp.asarray(f2))

    return (ner.reshape(B, S, nd),
            pos_out.reshape(B, P, 2),
            re_out.reshape(B, P, re_dim))


# speedup vs baseline: 3.5182x; 1.0035x over previous
import functools

import numpy as np

import jax
import jax.numpy as jnp
from jax import lax
from jax.experimental import pallas as pl
from jax.experimental.pallas import tpu as pltpu
from jax.experimental.pallas import tpu_sc as plsc

_PREC = lax.Precision.DEFAULT


def _dot(a, b):
    return jnp.dot(a, b, preferred_element_type=jnp.float32, precision=_PREC)


def _sc_gather(x2, idx_flat, ent_lim, seq_len):
    n_idx = idx_flat.shape[0]
    d = x2.shape[1]
    rows_per_w = 16
    n_workers = n_idx // rows_per_w

    mesh = plsc.VectorSubcoreMesh(core_axis_name="c", subcore_axis_name="s")

    @functools.partial(
        pl.kernel,
        mesh=mesh,
        out_type=jax.ShapeDtypeStruct((n_idx, d), jnp.float32),
        scratch_types=[
            pltpu.VMEM((rows_per_w,), jnp.int32),
            pltpu.VMEM((rows_per_w, d), jnp.float32),
            pltpu.SemaphoreType.DMA,
        ],
    )
    def _gather(hid_hbm, idx_hbm, out_hbm, idx_v, rows_v, sem):
        wid = lax.axis_index("s") * 2 + lax.axis_index("c")

        @pl.when(wid < n_workers)
        def _():
            base = wid * rows_per_w
            pltpu.sync_copy(idx_hbm.at[pl.ds(base, rows_per_w)], idx_v)
            g = lax.iota(jnp.int32, 16) + base
            scale = (1 << 16) // ent_lim + 1
            b = lax.shift_right_logical(g * scale, 16)
            idx_v[...] = idx_v[...] + b * seq_len + 1
            pltpu.async_copy(hid_hbm.at[idx_v], rows_v, sem).wait()
            pltpu.sync_copy(rows_v, out_hbm.at[pl.ds(base, rows_per_w)])

    return _gather(x2, idx_flat)


def _tc_body(n_ner_steps, re_dim, ent_lim,
             x_ref, wn1_ref, bn1_ref, wn2_ref, bn2_ref,
             ents_ref, posf_ref,
             wh1a_ref, wh1b_ref, bh1_ref, wh2_ref, bh2_ref,
             wt1a_ref, wt1b_ref, bt1_ref, wt2_ref, bt2_ref,
             wbil_ref, bbil_ref, wlh_ref, wlt_ref,
             rall_ref, call_ref, mbd_ref, ffold_ref, mt_ref, f2_ref,
             ner_ref, re_out_ref, pos_out_ref):
    hblk = jnp.maximum(_dot(x_ref[...], wn1_ref[...]) + bn1_ref[...], 0.0)
    ner_ref[...] = _dot(hblk, wn2_ref[...]) + bn2_ref[...]

    @pl.when(pl.program_id(0) == n_ner_steps)
    def _re_head():
        x = ents_ref[...]
        hn = jnp.maximum(_dot(x, wn1_ref[...]) + bn1_ref[...], 0.0)
        nerr = _dot(hn, wn2_ref[...]) + bn2_ref[...]
        p = jax.nn.softmax(nerr, axis=-1)

        h = jnp.maximum(_dot(x, wh1a_ref[...]) + _dot(p, wh1b_ref[...])
                        + bh1_ref[...], 0.0)
        head = _dot(h, wh2_ref[...]) + bh2_ref[...]
        t = jnp.maximum(_dot(x, wt1a_ref[...]) + _dot(p, wt1b_ref[...])
                        + bt1_ref[...], 0.0)
        tail = _dot(t, wt2_ref[...]) + bt2_ref[...]

        mbd = mbd_ref[...]
        ffold = ffold_ref[...]
        cols = []
        for o in range(re_dim):
            a_o = _dot(head, wbil_ref[o])
            g_o = lax.dot_general(a_o, tail, (((1,), (1,)), ((), ())),
                                  preferred_element_type=jnp.float32,
                                  precision=_PREC)
            cols.append(_dot(g_o * mbd, ffold))
        bilp = jnp.concatenate(cols, axis=1) + bbil_ref[...]

        head_l = _dot(head, wlh_ref[...])
        tail_l = _dot(tail, wlt_ref[...])

        rall = rall_ref[...]
        callm = call_ref[...]
        s1 = _dot(rall, bilp) * mt_ref[...]
        bilsel = _dot(s1, f2_ref[...])
        linsel = _dot(rall, head_l) + _dot(callm, tail_l)
        re_out_ref[...] = bilsel + linsel

        posf = posf_ref[...]
        ph = jnp.dot(rall, posf, preferred_element_type=jnp.float32,
                     precision=lax.Precision.HIGHEST)
        pt = jnp.dot(callm, posf, preferred_element_type=jnp.float32,
                     precision=lax.Precision.HIGHEST)
        pos_out_ref[...] = jnp.concatenate([ph, pt], axis=1).astype(jnp.int32)


def kernel(hidden, entity_idx, W_ner1, b_ner1, W_ner2, b_ner2,
           W_h1, b_h1, W_h2, b_h2, W_t1, b_t1, W_t2, b_t2,
           W_bil, b_bil, W_lin):
    B, S, D = hidden.shape
    L = entity_idx.shape[1]
    nd = W_ner2.shape[1]
    h_dim = W_h2.shape[0]
    re_dim = W_bil.shape[0]
    P = L * (L - 1)
    BL = B * L
    BP = B * P

    x2 = hidden.reshape(B * S, D)

    ents = _sc_gather(x2, entity_idx.reshape(-1), L, S)

    prs = [(i, j) for i in range(L) for j in range(L) if j != i]
    ih = np.array([q[0] for q in prs], dtype=np.int64)
    it = np.array([q[1] for q in prs], dtype=np.int64)

    rall = np.zeros((BP, BL), dtype=np.float32)
    callm = np.zeros((BP, BL), dtype=np.float32)
    bb = np.repeat(np.arange(B), P) * L
    rall[np.arange(BP), bb + np.tile(ih, B)] = 1.0
    callm[np.arange(BP), bb + np.tile(it, B)] = 1.0

    xg = np.arange(BL) // L
    mbd = (xg[:, None] == xg[None, :]).astype(np.float32)
    ffold = (np.arange(BL)[:, None] % L ==
             np.arange(L)[None, :]).astype(np.float32)
    mt90 = np.zeros((P, re_dim * L), dtype=np.float32)
    mt90[np.arange(P)[:, None],
         np.arange(re_dim)[None, :] * L + it[:, None]] = 1.0
    mt = np.tile(mt90, (B, 1))
    f2 = (np.arange(re_dim * L)[:, None] // L ==
          np.arange(re_dim)[None, :]).astype(np.float32)

    posf = (entity_idx.reshape(BL, 1) + 1).astype(jnp.float32)
    bbil_exp = jnp.repeat(b_bil, L).reshape(1, re_dim * L)

    block_m = 1024
    n_ner_steps = (B * S) // block_m
    grid = (n_ner_steps + 1,)
    body = functools.partial(_tc_body, n_ner_steps, re_dim, L)

    def _xmap(i):
        return (jnp.minimum(i, n_ner_steps - 1), 0)

    _const = lambda i: (0, 0)
    _const3 = lambda i: (0, 0, 0)

    ner, re_out, pos_out = pl.pallas_call(
        body,
        grid=grid,
        in_specs=[
            pl.BlockSpec((block_m, D), _xmap),
            pl.BlockSpec((D, D), _const),
            pl.BlockSpec((1, D), _const),
            pl.BlockSpec((D, nd), _const),
            pl.BlockSpec((1, nd), _const),
            pl.BlockSpec((BL, D), _const),
            pl.BlockSpec((BL, 1), _const),
            pl.BlockSpec((D, h_dim), _const),
            pl.BlockSpec((nd, h_dim), _const),
            pl.BlockSpec((1, h_dim), _const),
            pl.BlockSpec((h_dim, h_dim), _const),
            pl.BlockSpec((1, h_dim), _const),
            pl.BlockSpec((D, h_dim), _const),
            pl.BlockSpec((nd, h_dim), _const),
            pl.BlockSpec((1, h_dim), _const),
            pl.BlockSpec((h_dim, h_dim), _const),
            pl.BlockSpec((1, h_dim), _const),
            pl.BlockSpec((re_dim, h_dim, h_dim), _const3),
            pl.BlockSpec((1, re_dim * L), _const),
            pl.BlockSpec((h_dim, re_dim), _const),
            pl.BlockSpec((h_dim, re_dim), _const),
            pl.BlockSpec((BP, BL), _const),
            pl.BlockSpec((BP, BL), _const),
            pl.BlockSpec((BL, BL), _const),
            pl.BlockSpec((BL, L), _const),
            pl.BlockSpec((BP, re_dim * L), _const),
            pl.BlockSpec((re_dim * L, re_dim), _const),
        ],
        out_specs=(
            pl.BlockSpec((block_m, nd), _xmap),
            pl.BlockSpec((BP, re_dim), _const),
            pl.BlockSpec((BP, 2), _const),
        ),
        out_shape=(
            jax.ShapeDtypeStruct((B * S, nd), jnp.float32),
            jax.ShapeDtypeStruct((BP, re_dim), jnp.float32),
            jax.ShapeDtypeStruct((BP, 2), jnp.int32),
        ),
    )(x2, W_ner1, b_ner1.reshape(1, D), W_ner2, b_ner2.reshape(1, nd),
      ents, posf,
      W_h1[:D], W_h1[D:], b_h1.reshape(1, h_dim), W_h2, b_h2.reshape(1, h_dim),
      W_t1[:D], W_t1[D:], b_t1.reshape(1, h_dim), W_t2, b_t2.reshape(1, h_dim),
      W_bil, bbil_exp, W_lin[:h_dim], W_lin[h_dim:],
      jnp.asarray(rall), jnp.asarray(callm), jnp.asarray(mbd),
      jnp.asarray(ffold), jnp.asarray(mt), jnp.asarray(f2))

    return (ner.reshape(B, S, nd),
            pos_out.reshape(B, P, 2),
            re_out.reshape(B, P, re_dim))
